# baseline (speedup 1.0000x reference)
import jax
import jax.numpy as jnp
from jax.experimental import pallas as pl
from jax.experimental.pallas import tpu as pltpu

_BN = 128


def _acke_body(x_ref, nw_ref, ow_ref, o1_ref, o2_ref):
    x = x_ref[...]
    dims = (((1,), (1,)), ((), ()))
    o1_ref[...] = jax.lax.dot_general(
        x, nw_ref[...], dims, preferred_element_type=jnp.float32)
    o2_ref[...] = jax.lax.dot_general(
        x, ow_ref[...], dims, preferred_element_type=jnp.float32)


@jax.jit
def kernel(x, new_weight, orig_weight):
    b, k = x.shape
    n = new_weight.shape[0]
    grid = (n // _BN,)
    out_shape = jax.ShapeDtypeStruct((b, n), jnp.float32)
    call = pl.pallas_call(
        _acke_body,
        grid=grid,
        in_specs=[
            pl.BlockSpec((b, k), lambda j: (0, 0)),
            pl.BlockSpec((_BN, k), lambda j: (j, 0)),
            pl.BlockSpec((_BN, k), lambda j: (j, 0)),
        ],
        out_specs=[
            pl.BlockSpec((b, _BN), lambda j: (0, j)),
            pl.BlockSpec((b, _BN), lambda j: (0, j)),
        ],
        out_shape=[out_shape, out_shape],
        compiler_params=pltpu.CompilerParams(
            dimension_semantics=("arbitrary",)),
    )
    layer_out, original_layer_output = call(x, new_weight, orig_weight)
    return (layer_out, original_layer_output)


# speedup vs baseline: 1.1628x; 1.1628x over previous
import jax
import jax.numpy as jnp
from jax.experimental import pallas as pl
from jax.experimental.pallas import tpu as pltpu

_BN = 256
_KH = 2048


def _acke_body(x_ref, nw0_ref, nw1_ref, ow0_ref, ow1_ref, o1_ref, o2_ref):
    xl = x_ref[:, :_KH]
    xh = x_ref[:, _KH:]
    dims = (((1,), (1,)), ((), ()))
    o1_ref[...] = (
        jax.lax.dot_general(xl, nw0_ref[...], dims,
                            preferred_element_type=jnp.float32)
        + jax.lax.dot_general(xh, nw1_ref[...], dims,
                              preferred_element_type=jnp.float32))
    o2_ref[...] = (
        jax.lax.dot_general(xl, ow0_ref[...], dims,
                            preferred_element_type=jnp.float32)
        + jax.lax.dot_general(xh, ow1_ref[...], dims,
                              preferred_element_type=jnp.float32))


@jax.jit
def kernel(x, new_weight, orig_weight):
    b, k = x.shape
    n = new_weight.shape[0]
    grid = (n // _BN,)
    out_shape = jax.ShapeDtypeStruct((b, n), jnp.float32)
    call = pl.pallas_call(
        _acke_body,
        grid=grid,
        in_specs=[
            pl.BlockSpec((b, k), lambda j: (0, 0)),
            pl.BlockSpec((_BN, _KH), lambda j: (j, 0)),
            pl.BlockSpec((_BN, _KH), lambda j: (j, 1)),
            pl.BlockSpec((_BN, _KH), lambda j: (j, 0)),
            pl.BlockSpec((_BN, _KH), lambda j: (j, 1)),
        ],
        out_specs=[
            pl.BlockSpec((b, _BN), lambda j: (0, j)),
            pl.BlockSpec((b, _BN), lambda j: (0, j)),
        ],
        out_shape=[out_shape, out_shape],
        compiler_params=pltpu.CompilerParams(
            dimension_semantics=("arbitrary",)),
    )
    layer_out, original_layer_output = call(
        x, new_weight, new_weight, orig_weight, orig_weight)
    return (layer_out, original_layer_output)


# speedup vs baseline: 1.2059x; 1.0371x over previous
import jax
import jax.numpy as jnp
from jax.experimental import pallas as pl
from jax.experimental.pallas import tpu as pltpu

_BH = 128


def _acke_body(x_ref, nw0_ref, nw1_ref, ow0_ref, ow1_ref, o1_ref, o2_ref):
    x = x_ref[...]
    dims = (((1,), (1,)), ((), ()))
    o1_ref[:, :_BH] = jax.lax.dot_general(
        x, nw0_ref[...], dims, preferred_element_type=jnp.float32)
    o1_ref[:, _BH:] = jax.lax.dot_general(
        x, nw1_ref[...], dims, preferred_element_type=jnp.float32)
    o2_ref[:, :_BH] = jax.lax.dot_general(
        x, ow0_ref[...], dims, preferred_element_type=jnp.float32)
    o2_ref[:, _BH:] = jax.lax.dot_general(
        x, ow1_ref[...], dims, preferred_element_type=jnp.float32)


@jax.jit
def kernel(x, new_weight, orig_weight):
    b, k = x.shape
    n = new_weight.shape[0]
    grid = (n // (2 * _BH),)
    out_shape = jax.ShapeDtypeStruct((b, n), jnp.float32)
    call = pl.pallas_call(
        _acke_body,
        grid=grid,
        in_specs=[
            pl.BlockSpec((b, k), lambda j: (0, 0)),
            pl.BlockSpec((_BH, k), lambda j: (2 * j, 0)),
            pl.BlockSpec((_BH, k), lambda j: (2 * j + 1, 0)),
            pl.BlockSpec((_BH, k), lambda j: (2 * j, 0)),
            pl.BlockSpec((_BH, k), lambda j: (2 * j + 1, 0)),
        ],
        out_specs=[
            pl.BlockSpec((b, 2 * _BH), lambda j: (0, j)),
            pl.BlockSpec((b, 2 * _BH), lambda j: (0, j)),
        ],
        out_shape=[out_shape, out_shape],
        compiler_params=pltpu.CompilerParams(
            dimension_semantics=("arbitrary",)),
    )
    layer_out, original_layer_output = call(
        x, new_weight, new_weight, orig_weight, orig_weight)
    return (layer_out, original_layer_output)
